# baseline (speedup 1.0000x reference)
import functools

import jax
import jax.numpy as jnp
from jax import lax
from jax.experimental import pallas as pl
from jax.experimental.pallas import tpu as pltpu
from jax.experimental.pallas import tpu_sc as plsc

_VOCAB = 10001
_EMB = 100
_NCLS = 2
_B = 4096
_L = 200

_VP = 10016
_BSLOT = 10008

_info = plsc.get_sparse_core_info()
_NC, _NS = _info.num_cores, _info.num_subcores
_NW = _NC * _NS
_ROWS = _B // _NW
_WORDS = _ROWS // 2
_GROUPS = _WORDS // 16
_JUNROLL = 10


def _round_bf16_bits(u):
    return (u + 0x7FFF + ((u >> 16) & 1)) & jnp.uint32(0xFFFF0000)


def _p_body(tab_ref, w_ref, b_ref, p_ref):
    w = w_ref[...] * (1.0 / _L)
    p = lax.dot_general(w, tab_ref[...], (((1,), (1,)), ((), ())),
                        preferred_element_type=jnp.float32)
    p = jnp.concatenate([p, jnp.zeros((_NCLS, _VP - _VOCAB), jnp.float32)],
                        axis=1)
    col = lax.broadcasted_iota(jnp.int32, (_NCLS, _VP), 1)
    p = jnp.where(col == 0, 0.0, p)
    p = jnp.where(col == _BSLOT, b_ref[...], p)
    u = lax.bitcast_convert_type(p, jnp.uint32)
    hi = _round_bf16_bits(u[0:1, :])
    lo = _round_bf16_bits(u[1:2, :]) >> 16
    p_ref[...] = lax.bitcast_convert_type(hi | lo, jnp.int32)


def _sc_body(xt_hbm, p_hbm, out_hbm, xbuf, pp, obuf, sem_x, sem_p):
    wid = lax.axis_index("s") * _NC + lax.axis_index("c")
    cp_x = pltpu.async_copy(xt_hbm.at[:, pl.ds(wid * _WORDS, _WORDS)], xbuf,
                            sem_x)
    cp_p = pltpu.async_copy(p_hbm.at[0], pp, sem_p)
    cp_x.wait()
    cp_p.wait()

    hi_mask = jnp.full((16,), -65536, jnp.int32)

    def unpack(pk):
        return (plsc.bitcast(pk & hi_mask, jnp.float32),
                plsc.bitcast(pk << 16, jnp.float32))

    def group(g, carry0):
        wloc = g * 16 + lax.iota(jnp.int32, 16)
        bv = plsc.load_gather(pp, [jnp.full((16,), _BSLOT, jnp.int32)])
        b0, b1 = unpack(bv)
        accs = (b0, b1, b0, b1)

        def chunk(jc, carry):
            a0e, a1e, a0o, a1o = carry
            for k in range(_JUNROLL):
                j = jc * _JUNROLL + k
                xw = plsc.load_gather(xbuf, [jnp.full((16,), j, jnp.int32),
                                             wloc])
                xe = xw & 0xFFFF
                xo = lax.shift_right_logical(xw, 16)
                h0, l0 = unpack(plsc.load_gather(pp, [xe]))
                h1, l1 = unpack(plsc.load_gather(pp, [xo]))
                a0e, a1e = a0e + h0, a1e + l0
                a0o, a1o = a0o + h1, a1o + l1
            return a0e, a1e, a0o, a1o

        a0e, a1e, a0o, a1o = lax.fori_loop(0, _L // _JUNROLL, chunk, accs)
        reven = wloc * 2
        zv = jnp.zeros((16,), jnp.int32)
        plsc.store_scatter(obuf, [reven, zv], a0e)
        plsc.store_scatter(obuf, [reven, zv + 1], a1e)
        plsc.store_scatter(obuf, [reven + 1, zv], a0o)
        plsc.store_scatter(obuf, [reven + 1, zv + 1], a1o)
        return carry0

    lax.fori_loop(0, _GROUPS, group, 0)
    pltpu.sync_copy(obuf, out_hbm.at[pl.ds(wid * _ROWS, _ROWS)])


def kernel(x, table, W, b):
    p = pl.pallas_call(
        _p_body,
        out_shape=jax.ShapeDtypeStruct((1, _VP), jnp.int32),
    )(table, W, b.reshape(_NCLS, 1))

    xt = lax.bitcast_convert_type(
        x.astype(jnp.int16).T.reshape(_L, _B // 2, 2), jnp.int32)

    mesh = plsc.VectorSubcoreMesh(core_axis_name="c", subcore_axis_name="s")
    sc = functools.partial(
        pl.kernel,
        mesh=mesh,
        out_type=jax.ShapeDtypeStruct((_B, _NCLS), jnp.float32),
        scratch_types=[
            pltpu.VMEM((_L, _WORDS), jnp.int32),
            pltpu.VMEM((_VP,), jnp.int32),
            pltpu.VMEM((_ROWS, _NCLS), jnp.float32),
            pltpu.SemaphoreType.DMA,
            pltpu.SemaphoreType.DMA,
        ],
        compiler_params=pltpu.CompilerParams(
            needs_layout_passes=False, use_tc_tiling_on_sc=False),
    )(_sc_body)
    return sc(xt, p)


# speedup vs baseline: 1.0838x; 1.0838x over previous
import functools

import jax
import jax.numpy as jnp
from jax import lax
from jax.experimental import pallas as pl
from jax.experimental.pallas import tpu as pltpu
from jax.experimental.pallas import tpu_sc as plsc

_VOCAB = 10001
_EMB = 100
_NCLS = 2
_B = 4096
_L = 200

_VP = 10016
_BSLOT = 10008

_info = plsc.get_sparse_core_info()
_NC, _NS = _info.num_cores, _info.num_subcores
_NW = _NC * _NS
_ROWS = _B // _NW
_WORDS = _ROWS // 2
_GROUPS = _WORDS // 16
_JUNROLL = 10


def _round_bf16_bits(u):
    return (u + 0x7FFF + ((u >> 16) & 1)) & jnp.uint32(0xFFFF0000)


def _p_body(tab_ref, w_ref, b_ref, p_ref):
    w = w_ref[...] * (1.0 / _L)
    p = lax.dot_general(w, tab_ref[...], (((1,), (1,)), ((), ())),
                        preferred_element_type=jnp.float32)
    p = jnp.concatenate([p, jnp.zeros((_NCLS, _VP - _VOCAB), jnp.float32)],
                        axis=1)
    col = lax.broadcasted_iota(jnp.int32, (_NCLS, _VP), 1)
    p = jnp.where(col == 0, 0.0, p)
    p = jnp.where(col == _BSLOT, b_ref[...], p)
    u = lax.bitcast_convert_type(p, jnp.uint32)
    hi = _round_bf16_bits(u[0:1, :])
    lo = _round_bf16_bits(u[1:2, :]) >> 16
    p_ref[...] = lax.bitcast_convert_type(hi | lo, jnp.int32)


def _sc_body(xt_hbm, p_hbm, out_hbm, xbuf, pp, obuf, sem_x, sem_p):
    wid = lax.axis_index("s") * _NC + lax.axis_index("c")
    cp_x = pltpu.async_copy(xt_hbm.at[:, pl.ds(wid * _WORDS, _WORDS)], xbuf,
                            sem_x)
    cp_p = pltpu.async_copy(p_hbm.at[0], pp, sem_p)
    cp_x.wait()
    cp_p.wait()

    hi_mask = jnp.full((16,), -65536, jnp.int32)

    def unpack(pk):
        return (plsc.bitcast(pk & hi_mask, jnp.float32),
                plsc.bitcast(pk << 16, jnp.float32))

    def group(g, carry0):
        wloc = g * 16 + lax.iota(jnp.int32, 16)
        bv = plsc.load_gather(pp, [jnp.full((16,), _BSLOT, jnp.int32)])
        b0, b1 = unpack(bv)
        accs = (b0, b1, b0, b1)

        def chunk(jc, carry):
            a0e, a1e, a0o, a1o = carry
            for k in range(_JUNROLL):
                j = jc * _JUNROLL + k
                xw = plsc.load_gather(xbuf, [jnp.full((16,), j, jnp.int32),
                                             wloc])
                xe = xw & 0xFFFF
                xo = lax.shift_right_logical(xw, 16)
                h0, l0 = unpack(plsc.load_gather(pp, [xe]))
                h1, l1 = unpack(plsc.load_gather(pp, [xo]))
                a0e, a1e = a0e + h0, a1e + l0
                a0o, a1o = a0o + h1, a1o + l1
            return a0e, a1e, a0o, a1o

        a0e, a1e, a0o, a1o = lax.fori_loop(0, _L // _JUNROLL, chunk, accs)
        reven = wloc * 2
        zv = jnp.zeros((16,), jnp.int32)
        plsc.store_scatter(obuf, [reven, zv], a0e)
        plsc.store_scatter(obuf, [reven, zv + 1], a1e)
        plsc.store_scatter(obuf, [reven + 1, zv], a0o)
        plsc.store_scatter(obuf, [reven + 1, zv + 1], a1o)
        return carry0

    lax.fori_loop(0, _GROUPS, group, 0)
    pltpu.sync_copy(obuf, out_hbm.at[pl.ds(wid * _ROWS, _ROWS)])


def kernel(x, table, W, b):
    p = pl.pallas_call(
        _p_body,
        out_shape=jax.ShapeDtypeStruct((1, _VP), jnp.int32),
    )(table, W, b.reshape(_NCLS, 1))

    xr = x.reshape(_B // 2, 2, _L)
    xt = (xr[:, 0, :] | (xr[:, 1, :] << 16)).T

    mesh = plsc.VectorSubcoreMesh(core_axis_name="c", subcore_axis_name="s")
    sc = functools.partial(
        pl.kernel,
        mesh=mesh,
        out_type=jax.ShapeDtypeStruct((_B, _NCLS), jnp.float32),
        scratch_types=[
            pltpu.VMEM((_L, _WORDS), jnp.int32),
            pltpu.VMEM((_VP,), jnp.int32),
            pltpu.VMEM((_ROWS, _NCLS), jnp.float32),
            pltpu.SemaphoreType.DMA,
            pltpu.SemaphoreType.DMA,
        ],
        compiler_params=pltpu.CompilerParams(
            needs_layout_passes=False, use_tc_tiling_on_sc=False),
    )(_sc_body)
    return sc(xt, p)


# speedup vs baseline: 1.2110x; 1.1174x over previous
import functools

import jax
import jax.numpy as jnp
from jax import lax
from jax.experimental import pallas as pl
from jax.experimental.pallas import tpu as pltpu
from jax.experimental.pallas import tpu_sc as plsc

_VOCAB = 10001
_EMB = 100
_NCLS = 2
_B = 4096
_L = 200

_VP = 10016
_BSLOT = 10008

_info = plsc.get_sparse_core_info()
_NC, _NS = _info.num_cores, _info.num_subcores
_NW = _NC * _NS
_ROWS = _B // _NW
_GROUPS = _ROWS // 16
_XSTRIDE = _L + 1
_JUNROLL = 10


def _round_bf16_bits(u):
    return (u + 0x7FFF + ((u >> 16) & 1)) & jnp.uint32(0xFFFF0000)


def _p_body(tab_ref, w_ref, b_ref, p_ref):
    w = w_ref[...] * (1.0 / _L)
    p = lax.dot_general(w, tab_ref[...], (((1,), (1,)), ((), ())),
                        preferred_element_type=jnp.float32)
    p = jnp.concatenate([p, jnp.zeros((_NCLS, _VP - _VOCAB), jnp.float32)],
                        axis=1)
    col = lax.broadcasted_iota(jnp.int32, (_NCLS, _VP), 1)
    p = jnp.where(col == 0, 0.0, p)
    p = jnp.where(col == _BSLOT, b_ref[...], p)
    u = lax.bitcast_convert_type(p, jnp.uint32)
    hi = _round_bf16_bits(u[0:1, :])
    lo = _round_bf16_bits(u[1:2, :]) >> 16
    p_ref[...] = lax.bitcast_convert_type(hi | lo, jnp.int32)


def _sc_body(x_hbm, p_hbm, out_hbm, xbuf, pp, obuf, sem_x, sem_p):
    wid = lax.axis_index("s") * _NC + lax.axis_index("c")
    cp_x = pltpu.async_copy(x_hbm.at[pl.ds(wid * _ROWS, _ROWS)],
                            xbuf.at[:, pl.ds(0, _L)], sem_x)
    cp_p = pltpu.async_copy(p_hbm.at[0], pp, sem_p)
    cp_x.wait()
    cp_p.wait()

    hi_mask = jnp.full((16,), -65536, jnp.int32)

    def unpack(pk):
        return (plsc.bitcast(pk & hi_mask, jnp.float32),
                plsc.bitcast(pk << 16, jnp.float32))

    def group(g, carry0):
        rloc = g * 16 + lax.iota(jnp.int32, 16)
        bv = plsc.load_gather(pp, [jnp.full((16,), _BSLOT, jnp.int32)])
        acc0, acc1 = unpack(bv)

        def chunk(jc, carry):
            a0, a1 = carry
            for k in range(_JUNROLL):
                j = jc * _JUNROLL + k
                xv = plsc.load_gather(xbuf, [rloc, jnp.full((16,), j,
                                                            jnp.int32)])
                h, l = unpack(plsc.load_gather(pp, [xv]))
                a0, a1 = a0 + h, a1 + l
            return a0, a1

        acc0, acc1 = lax.fori_loop(0, _L // _JUNROLL, chunk, (acc0, acc1))
        zv = jnp.zeros((16,), jnp.int32)
        plsc.store_scatter(obuf, [rloc, zv], acc0)
        plsc.store_scatter(obuf, [rloc, zv + 1], acc1)
        return carry0

    lax.fori_loop(0, _GROUPS, group, 0)
    pltpu.sync_copy(obuf, out_hbm.at[pl.ds(wid * _ROWS, _ROWS)])


def kernel(x, table, W, b):
    p = pl.pallas_call(
        _p_body,
        out_shape=jax.ShapeDtypeStruct((1, _VP), jnp.int32),
    )(table, W, b.reshape(_NCLS, 1))

    mesh = plsc.VectorSubcoreMesh(core_axis_name="c", subcore_axis_name="s")
    sc = functools.partial(
        pl.kernel,
        mesh=mesh,
        out_type=jax.ShapeDtypeStruct((_B, _NCLS), jnp.float32),
        scratch_types=[
            pltpu.VMEM((_ROWS, _XSTRIDE), jnp.int32),
            pltpu.VMEM((_VP,), jnp.int32),
            pltpu.VMEM((_ROWS, _NCLS), jnp.float32),
            pltpu.SemaphoreType.DMA,
            pltpu.SemaphoreType.DMA,
        ],
        compiler_params=pltpu.CompilerParams(
            needs_layout_passes=False, use_tc_tiling_on_sc=False),
    )(_sc_body)
    return sc(x, p)


# speedup vs baseline: 1.5919x; 1.3145x over previous
import functools

import jax
import jax.numpy as jnp
from jax import lax
from jax.experimental import pallas as pl
from jax.experimental.pallas import tpu as pltpu
from jax.experimental.pallas import tpu_sc as plsc

_VOCAB = 10001
_EMB = 100
_NCLS = 2
_B = 4096
_L = 200

_VBLK = 1280
_VGRID = 8
_VP = _VBLK * _VGRID
_BSLOT = 10232

_info = plsc.get_sparse_core_info()
_NC, _NS = _info.num_cores, _info.num_subcores
_NW = _NC * _NS
_ROWS = _B // _NW
_GROUPS = _ROWS // 16
_JUNROLL = 20


def _round_bf16_bits(u):
    return (u + 0x7FFF + ((u >> 16) & 1)) & jnp.uint32(0xFFFF0000)


def _p_body(tab_ref, w_ref, b_ref, p_ref):
    w = w_ref[...] * (1.0 / _L)
    p = lax.dot_general(w, tab_ref[...], (((1,), (1,)), ((), ())),
                        preferred_element_type=jnp.float32)
    col = (lax.broadcasted_iota(jnp.int32, (_NCLS, _VBLK), 1)
           + pl.program_id(0) * _VBLK)
    p = jnp.where((col == 0) | (col >= _VOCAB), 0.0, p)
    p = jnp.where(col == _BSLOT, b_ref[...], p)
    u = lax.bitcast_convert_type(p, jnp.uint32)
    hi = _round_bf16_bits(u[0:1, :])
    lo = _round_bf16_bits(u[1:2, :]) >> 16
    p_ref[...] = lax.bitcast_convert_type(hi | lo, jnp.int32)


def _sc_body(xt_hbm, p_hbm, out_hbm, xbuf, pp, obuf, sem_x, sem_p):
    wid = lax.axis_index("s") * _NC + lax.axis_index("c")
    base = wid * _ROWS
    cp_x = pltpu.async_copy(xt_hbm.at[:, pl.ds(base, _ROWS)], xbuf, sem_x)
    cp_p = pltpu.async_copy(p_hbm.at[0], pp, sem_p)
    cp_x.wait()
    cp_p.wait()

    hi_mask = jnp.full((16,), -65536, jnp.int32)

    def unpack(pk):
        return (plsc.bitcast(pk & hi_mask, jnp.float32),
                plsc.bitcast(pk << 16, jnp.float32))

    def group(g, carry0):
        rloc = g * 16 + lax.iota(jnp.int32, 16)
        bv = plsc.load_gather(pp, [jnp.full((16,), _BSLOT, jnp.int32)])
        acc0, acc1 = unpack(bv)

        def chunk(jc, carry):
            a0, a1 = carry
            for k in range(_JUNROLL):
                j = jc * _JUNROLL + k
                xv = plsc.load_gather(xbuf, [jnp.full((16,), j, jnp.int32),
                                             rloc])
                h, l = unpack(plsc.load_gather(pp, [xv]))
                a0, a1 = a0 + h, a1 + l
            return a0, a1

        acc0, acc1 = lax.fori_loop(0, _L // _JUNROLL, chunk, (acc0, acc1))
        zv = jnp.zeros((16,), jnp.int32)
        plsc.store_scatter(obuf, [zv, rloc], acc0)
        plsc.store_scatter(obuf, [zv + 1, rloc], acc1)
        return carry0

    lax.fori_loop(0, _GROUPS, group, 0)
    pltpu.sync_copy(obuf, out_hbm.at[:, pl.ds(base, _ROWS)])


def kernel(x, table, W, b):
    p = pl.pallas_call(
        _p_body,
        grid=(_VGRID,),
        in_specs=[
            pl.BlockSpec((_VBLK, _EMB), lambda i: (i, 0)),
            pl.BlockSpec((_NCLS, _EMB), lambda i: (0, 0)),
            pl.BlockSpec((_NCLS, 1), lambda i: (0, 0)),
        ],
        out_specs=pl.BlockSpec((1, _VBLK), lambda i: (0, i)),
        out_shape=jax.ShapeDtypeStruct((1, _VP), jnp.int32),
    )(table, W, b.reshape(_NCLS, 1))

    mesh = plsc.VectorSubcoreMesh(core_axis_name="c", subcore_axis_name="s")
    sc = functools.partial(
        pl.kernel,
        mesh=mesh,
        out_type=jax.ShapeDtypeStruct((_NCLS, _B), jnp.float32),
        scratch_types=[
            pltpu.VMEM((_L, _ROWS), jnp.int32),
            pltpu.VMEM((_VP,), jnp.int32),
            pltpu.VMEM((_NCLS, _ROWS), jnp.float32),
            pltpu.SemaphoreType.DMA,
            pltpu.SemaphoreType.DMA,
        ],
        compiler_params=pltpu.CompilerParams(
            needs_layout_passes=False, use_tc_tiling_on_sc=False,
            disable_bounds_checks=True),
    )(_sc_body)
    return sc(x.T, p).T


# speedup vs baseline: 1.7338x; 1.0891x over previous
import functools

import jax
import jax.numpy as jnp
from jax import lax
from jax.experimental import pallas as pl
from jax.experimental.pallas import tpu as pltpu
from jax.experimental.pallas import tpu_sc as plsc

_VOCAB = 10001
_EMB = 100
_NCLS = 2
_B = 4096
_L = 200

_VBLK = 1280
_VGRID = 8
_VP = _VBLK * _VGRID
_BSLOT = 10232

_info = plsc.get_sparse_core_info()
_NC, _NS = _info.num_cores, _info.num_subcores
_NW = _NC * _NS
_ROWS = _B // _NW
_GROUPS = _ROWS // 16
_JUNROLL = 20


def _round_bf16_bits(u):
    return (u + 0x7FFF + ((u >> 16) & 1)) & jnp.uint32(0xFFFF0000)


def _p_body(tab_ref, w_ref, b_ref, p_ref):
    w = w_ref[...] * (1.0 / _L)
    p = lax.dot_general(w, tab_ref[...], (((1,), (1,)), ((), ())),
                        preferred_element_type=jnp.float32)
    p = jnp.concatenate([p, jnp.zeros((_NCLS, _VP - _VOCAB), jnp.float32)],
                        axis=1)
    col = lax.broadcasted_iota(jnp.int32, (_NCLS, _VP), 1)
    p = jnp.where(col == 0, 0.0, p)
    p = jnp.where(col == _BSLOT, b_ref[...], p)
    u = lax.bitcast_convert_type(p, jnp.uint32)
    hi = _round_bf16_bits(u[0:1, :])
    lo = _round_bf16_bits(u[1:2, :]) >> 16
    p_ref[...] = lax.bitcast_convert_type(hi | lo, jnp.int32)


def _sc_body(xt_hbm, p_hbm, out_hbm, xbuf, pp, obuf, sem_x, sem_p):
    wid = lax.axis_index("s") * _NC + lax.axis_index("c")
    base = wid * _ROWS
    cp_x = pltpu.async_copy(xt_hbm.at[wid], xbuf, sem_x)
    cp_p = pltpu.async_copy(p_hbm.at[0], pp, sem_p)
    cp_x.wait()
    cp_p.wait()

    hi_mask = jnp.full((16,), -65536, jnp.int32)

    def unpack(pk):
        return (plsc.bitcast(pk & hi_mask, jnp.float32),
                plsc.bitcast(pk << 16, jnp.float32))

    def group(g, carry0):
        rloc = g * 16 + lax.iota(jnp.int32, 16)
        bv = plsc.load_gather(pp, [jnp.full((16,), _BSLOT, jnp.int32)])
        acc0, acc1 = unpack(bv)

        def chunk(jc, carry):
            a0, a1 = carry
            for k in range(_JUNROLL):
                j = jc * _JUNROLL + k
                xv = plsc.load_gather(xbuf, [jnp.full((16,), j, jnp.int32),
                                             rloc])
                h, l = unpack(plsc.load_gather(pp, [xv]))
                a0, a1 = a0 + h, a1 + l
            return a0, a1

        acc0, acc1 = lax.fori_loop(0, _L // _JUNROLL, chunk, (acc0, acc1))
        zv = jnp.zeros((16,), jnp.int32)
        plsc.store_scatter(obuf, [zv, rloc], acc0)
        plsc.store_scatter(obuf, [zv + 1, rloc], acc1)
        return carry0

    lax.fori_loop(0, _GROUPS, group, 0)
    pltpu.sync_copy(obuf, out_hbm.at[:, pl.ds(base, _ROWS)])


def kernel(x, table, W, b):
    p = pl.pallas_call(
        _p_body,
        out_shape=jax.ShapeDtypeStruct((1, _VP), jnp.int32),
    )(table, W, b.reshape(_NCLS, 1))

    xt = x.T.reshape(_L, _NW, _ROWS).transpose(1, 0, 2)

    mesh = plsc.VectorSubcoreMesh(core_axis_name="c", subcore_axis_name="s")
    sc = functools.partial(
        pl.kernel,
        mesh=mesh,
        out_type=jax.ShapeDtypeStruct((_NCLS, _B), jnp.float32),
        scratch_types=[
            pltpu.VMEM((_L, _ROWS), jnp.int32),
            pltpu.VMEM((_VP,), jnp.int32),
            pltpu.VMEM((_NCLS, _ROWS), jnp.float32),
            pltpu.SemaphoreType.DMA,
            pltpu.SemaphoreType.DMA,
        ],
        compiler_params=pltpu.CompilerParams(
            needs_layout_passes=False, use_tc_tiling_on_sc=False,
            disable_bounds_checks=True),
    )(_sc_body)
    return sc(xt, p).T


# speedup vs baseline: 1.7484x; 1.0084x over previous
import functools

import jax
import jax.numpy as jnp
from jax import lax
from jax.experimental import pallas as pl
from jax.experimental.pallas import tpu as pltpu
from jax.experimental.pallas import tpu_sc as plsc

_VOCAB = 10001
_EMB = 100
_NCLS = 2
_B = 4096
_L = 200

_VBLK = 1280
_VGRID = 8
_VP = _VBLK * _VGRID
_BSLOT = 10232

_info = plsc.get_sparse_core_info()
_NC, _NS = _info.num_cores, _info.num_subcores
_NW = _NC * _NS
_ROWS = _B // _NW
_GROUPS = _ROWS // 16
_JUNROLL = 20


def _round_bf16_bits(u):
    return (u + 0x7FFF + ((u >> 16) & 1)) & jnp.uint32(0xFFFF0000)


def _p_body(tab_ref, w_ref, b_ref, p_ref):
    blk = _VP // 2
    w = w_ref[...] * (1.0 / _L)
    p = lax.dot_general(w, tab_ref[...], (((1,), (1,)), ((), ())),
                        preferred_element_type=jnp.float32)
    col = (lax.broadcasted_iota(jnp.int32, (_NCLS, blk), 1)
           + pl.program_id(0) * blk)
    p = jnp.where((col == 0) | (col >= _VOCAB), 0.0, p)
    p = jnp.where(col == _BSLOT, b_ref[...], p)
    u = lax.bitcast_convert_type(p, jnp.uint32)
    hi = _round_bf16_bits(u[0:1, :])
    lo = _round_bf16_bits(u[1:2, :]) >> 16
    p_ref[...] = lax.bitcast_convert_type(hi | lo, jnp.int32)


def _sc_body(xt_hbm, p_hbm, out_hbm, xbuf, pp, obuf, sem_x, sem_p):
    wid = lax.axis_index("s") * _NC + lax.axis_index("c")
    base = wid * _ROWS
    cp_x = pltpu.async_copy(xt_hbm.at[wid], xbuf, sem_x)
    cp_p = pltpu.async_copy(p_hbm.at[0], pp, sem_p)
    cp_x.wait()
    cp_p.wait()

    hi_mask = jnp.full((16,), -65536, jnp.int32)

    def unpack(pk):
        return (plsc.bitcast(pk & hi_mask, jnp.float32),
                plsc.bitcast(pk << 16, jnp.float32))

    def group(g, carry0):
        rloc = g * 16 + lax.iota(jnp.int32, 16)
        bv = plsc.load_gather(pp, [jnp.full((16,), _BSLOT, jnp.int32)])
        acc0, acc1 = unpack(bv)

        def chunk(jc, carry):
            a0, a1 = carry
            for k in range(_JUNROLL):
                j = jc * _JUNROLL + k
                xv = plsc.load_gather(xbuf, [jnp.full((16,), j, jnp.int32),
                                             rloc])
                h, l = unpack(plsc.load_gather(pp, [xv]))
                a0, a1 = a0 + h, a1 + l
            return a0, a1

        acc0, acc1 = lax.fori_loop(0, _L // _JUNROLL, chunk, (acc0, acc1))
        zv = jnp.zeros((16,), jnp.int32)
        plsc.store_scatter(obuf, [zv, rloc], acc0)
        plsc.store_scatter(obuf, [zv + 1, rloc], acc1)
        return carry0

    lax.fori_loop(0, _GROUPS, group, 0)
    pltpu.sync_copy(obuf, out_hbm.at[:, pl.ds(base, _ROWS)])


def kernel(x, table, W, b):
    p = pl.pallas_call(
        _p_body,
        grid=(2,),
        in_specs=[
            pl.BlockSpec((_VP // 2, _EMB), lambda i: (i, 0)),
            pl.BlockSpec((_NCLS, _EMB), lambda i: (0, 0)),
            pl.BlockSpec((_NCLS, 1), lambda i: (0, 0)),
        ],
        out_specs=pl.BlockSpec((1, _VP // 2), lambda i: (0, i)),
        out_shape=jax.ShapeDtypeStruct((1, _VP), jnp.int32),
    )(table, W, b.reshape(_NCLS, 1))

    xt = x.T.reshape(_L, _NW, _ROWS).transpose(1, 0, 2)

    mesh = plsc.VectorSubcoreMesh(core_axis_name="c", subcore_axis_name="s")
    sc = functools.partial(
        pl.kernel,
        mesh=mesh,
        out_type=jax.ShapeDtypeStruct((_NCLS, _B), jnp.float32),
        scratch_types=[
            pltpu.VMEM((_L, _ROWS), jnp.int32),
            pltpu.VMEM((_VP,), jnp.int32),
            pltpu.VMEM((_NCLS, _ROWS), jnp.float32),
            pltpu.SemaphoreType.DMA,
            pltpu.SemaphoreType.DMA,
        ],
        compiler_params=pltpu.CompilerParams(
            needs_layout_passes=False, use_tc_tiling_on_sc=False,
            disable_bounds_checks=True),
    )(_sc_body)
    return sc(xt, p).T


# speedup vs baseline: 1.7525x; 1.0023x over previous
import functools

import jax
import jax.numpy as jnp
from jax import lax
from jax.experimental import pallas as pl
from jax.experimental.pallas import tpu as pltpu
from jax.experimental.pallas import tpu_sc as plsc

_VOCAB = 10001
_EMB = 100
_NCLS = 2
_B = 4096
_L = 200

_VP = 10240
_BSLOT = 10232

_info = plsc.get_sparse_core_info()
_NC, _NS = _info.num_cores, _info.num_subcores
_NW = _NC * _NS
_ROWS = _B // _NW
_GROUPS = _ROWS // 16
_JUNROLL = 20


def _round_bf16_bits(u):
    return (u + 0x7FFF + ((u >> 16) & 1)) & jnp.uint32(0xFFFF0000)


def _p_body(tab_ref, w_ref, b_ref, p_ref):
    blk = _VP // 2
    w = w_ref[...] * (1.0 / _L)
    p = lax.dot_general(w, tab_ref[...], (((1,), (1,)), ((), ())),
                        preferred_element_type=jnp.float32)
    col = (lax.broadcasted_iota(jnp.int32, (_NCLS, blk), 1)
           + pl.program_id(0) * blk)
    p = jnp.where((col == 0) | (col >= _VOCAB), 0.0, p)
    p = jnp.where(col == _BSLOT, b_ref[...], p)
    u = lax.bitcast_convert_type(p, jnp.uint32)
    hi = _round_bf16_bits(u[0:1, :])
    lo = _round_bf16_bits(u[1:2, :]) >> 16
    p_ref[...] = lax.bitcast_convert_type(hi | lo, jnp.int32)


def _sc_body(xt_hbm, p_hbm, out_hbm, xbuf, pp, obuf, sem_x, sem_p):
    wid = lax.axis_index("s") * _NC + lax.axis_index("c")
    base = wid * _ROWS
    cp_x = pltpu.async_copy(xt_hbm.at[wid], xbuf, sem_x)
    cp_p = pltpu.async_copy(p_hbm.at[0], pp, sem_p)
    cp_x.wait()
    cp_p.wait()

    hi_mask = jnp.full((16,), -65536, jnp.int32)

    def unpack(pk):
        return (plsc.bitcast(pk & hi_mask, jnp.float32),
                plsc.bitcast(pk << 16, jnp.float32))

    def group(g, carry0):
        rloc = g * 16 + lax.iota(jnp.int32, 16)
        bv = plsc.load_gather(pp, [jnp.full((16,), _BSLOT, jnp.int32)])
        acc0, acc1 = unpack(bv)

        def chunk(jc, carry):
            a0, a1 = carry
            for k in range(_JUNROLL):
                j = jc * _JUNROLL + k
                xv = plsc.load_gather(xbuf, [jnp.full((16,), j, jnp.int32),
                                             rloc])
                h, l = unpack(plsc.load_gather(pp, [xv]))
                a0, a1 = a0 + h, a1 + l
            return a0, a1

        acc0, acc1 = lax.fori_loop(0, _L // _JUNROLL, chunk, (acc0, acc1))
        zv = jnp.zeros((16,), jnp.int32)
        plsc.store_scatter(obuf, [zv, rloc], acc0)
        plsc.store_scatter(obuf, [zv + 1, rloc], acc1)
        return carry0

    lax.fori_loop(0, _GROUPS, group, 0)
    pltpu.sync_copy(obuf, out_hbm.at[:, pl.ds(base, _ROWS)])


def kernel(x, table, W, b):
    p = pl.pallas_call(
        _p_body,
        grid=(2,),
        in_specs=[
            pl.BlockSpec((_VP // 2, _EMB), lambda i: (i, 0)),
            pl.BlockSpec((_NCLS, _EMB), lambda i: (0, 0)),
            pl.BlockSpec((_NCLS, 1), lambda i: (0, 0)),
        ],
        out_specs=pl.BlockSpec((1, _VP // 2), lambda i: (0, i)),
        out_shape=jax.ShapeDtypeStruct((1, _VP), jnp.int32),
    )(table, W, b.reshape(_NCLS, 1))

    xt = x.T.reshape(_L, _NW, _ROWS).transpose(1, 0, 2)

    mesh = plsc.VectorSubcoreMesh(core_axis_name="c", subcore_axis_name="s")
    sc = functools.partial(
        pl.kernel,
        mesh=mesh,
        out_type=jax.ShapeDtypeStruct((_NCLS, _B), jnp.float32),
        scratch_types=[
            pltpu.VMEM((_L, _ROWS), jnp.int32),
            pltpu.VMEM((_VP,), jnp.int32),
            pltpu.VMEM((_NCLS, _ROWS), jnp.float32),
            pltpu.SemaphoreType.DMA,
            pltpu.SemaphoreType.DMA,
        ],
        compiler_params=pltpu.CompilerParams(
            needs_layout_passes=False, use_tc_tiling_on_sc=False,
            disable_bounds_checks=True),
    )(_sc_body)
    return sc(xt, p).T
